# baseline (speedup 1.0000x reference)
import jax
import jax.numpy as jnp
from jax.experimental import pallas as pl
from jax.experimental.pallas import tpu as pltpu


def _dma_body(x_ref, o_ref, sem):
    copy = pltpu.make_async_copy(x_ref, o_ref, sem)
    copy.start()
    copy.wait()


def kernel(x, embed_weight):
    del embed_weight
    flat = x.reshape(8192, 1024)
    out = pl.pallas_call(
        _dma_body,
        out_shape=jax.ShapeDtypeStruct(flat.shape, flat.dtype),
        in_specs=[pl.BlockSpec(memory_space=pl.ANY)],
        out_specs=pl.BlockSpec(memory_space=pl.ANY),
        scratch_shapes=[pltpu.SemaphoreType.DMA],
    )(flat)
    return out.reshape(x.shape)


# speedup vs baseline: 41.5450x; 41.5450x over previous
import jax
import jax.numpy as jnp
from jax.experimental import pallas as pl
from jax.experimental.pallas import tpu as pltpu


def _copy_body(x_ref, o_ref):
    o_ref[...] = x_ref[...]


def kernel(x, embed_weight):
    del embed_weight
    flat = x.reshape(8192, 1024)
    out = pl.pallas_call(
        _copy_body,
        out_shape=jax.ShapeDtypeStruct(flat.shape, flat.dtype),
        grid=(16,),
        in_specs=[pl.BlockSpec((512, 1024), lambda i: (i, 0))],
        out_specs=pl.BlockSpec((512, 1024), lambda i: (i, 0)),
        compiler_params=pltpu.CompilerParams(
            dimension_semantics=("parallel",),
        ),
    )(flat)
    return out.reshape(x.shape)


# speedup vs baseline: 49.0369x; 1.1803x over previous
import jax
import jax.numpy as jnp
from jax.experimental import pallas as pl
from jax.experimental.pallas import tpu as pltpu


def _copy_body(x_ref, o_ref):
    o_ref[...] = x_ref[...]


def kernel(x, embed_weight):
    del embed_weight
    flat = x.reshape(8192, 1024)
    out = pl.pallas_call(
        _copy_body,
        out_shape=jax.ShapeDtypeStruct(flat.shape, flat.dtype),
        grid=(4,),
        in_specs=[pl.BlockSpec((2048, 1024), lambda i: (i, 0))],
        out_specs=pl.BlockSpec((2048, 1024), lambda i: (i, 0)),
        compiler_params=pltpu.CompilerParams(
            dimension_semantics=("parallel",),
        ),
    )(flat)
    return out.reshape(x.shape)
